# baseline (speedup 1.0000x reference)
import jax
import jax.numpy as jnp
from jax.experimental import pallas as pl
from jax.experimental.pallas import tpu as pltpu

_ROWS = 8192
_COLS = 768
_CHUNKS = 8
_CHUNK_ROWS = _ROWS // _CHUNKS
_SLOTS = 4


def _copy_body(x_hbm, o_hbm, buf, in_sems, out_sems):
    def in_copy(c):
        return pltpu.make_async_copy(
            x_hbm.at[pl.ds(c * _CHUNK_ROWS, _CHUNK_ROWS), :],
            buf.at[c % _SLOTS],
            in_sems.at[c % _SLOTS],
        )

    def out_copy(c):
        return pltpu.make_async_copy(
            buf.at[c % _SLOTS],
            o_hbm.at[pl.ds(c * _CHUNK_ROWS, _CHUNK_ROWS), :],
            out_sems.at[c % _SLOTS],
        )

    for c in range(_SLOTS):
        in_copy(c).start()
    for c in range(_CHUNKS):
        if c >= 1 and c - 1 + _SLOTS < _CHUNKS:
            out_copy(c - 1).wait()
            in_copy(c - 1 + _SLOTS).start()
        in_copy(c).wait()
        out_copy(c).start()
    for c in range(max(_CHUNKS - _SLOTS, 0), _CHUNKS):
        out_copy(c).wait()


def kernel(prototypes):
    return pl.pallas_call(
        _copy_body,
        out_shape=jax.ShapeDtypeStruct((_ROWS, _COLS), prototypes.dtype),
        in_specs=[pl.BlockSpec(memory_space=pltpu.MemorySpace.HBM)],
        out_specs=pl.BlockSpec(memory_space=pltpu.MemorySpace.HBM),
        scratch_shapes=[
            pltpu.VMEM((_SLOTS, _CHUNK_ROWS, _COLS), jnp.float32),
            pltpu.SemaphoreType.DMA((_SLOTS,)),
            pltpu.SemaphoreType.DMA((_SLOTS,)),
        ],
    )(prototypes)


# speedup vs baseline: 1.0247x; 1.0247x over previous
import jax
import jax.numpy as jnp
from jax.experimental import pallas as pl
from jax.experimental.pallas import tpu as pltpu

_ROWS = 8192
_COLS = 768
_CHUNKS = 4
_CHUNK_ROWS = _ROWS // _CHUNKS
_SLOTS = 4


def _copy_body(x_hbm, o_hbm, buf, in_sems, out_sems):
    def in_copy(c):
        return pltpu.make_async_copy(
            x_hbm.at[pl.ds(c * _CHUNK_ROWS, _CHUNK_ROWS), :],
            buf.at[c % _SLOTS],
            in_sems.at[c % _SLOTS],
        )

    def out_copy(c):
        return pltpu.make_async_copy(
            buf.at[c % _SLOTS],
            o_hbm.at[pl.ds(c * _CHUNK_ROWS, _CHUNK_ROWS), :],
            out_sems.at[c % _SLOTS],
        )

    for c in range(_SLOTS):
        in_copy(c).start()
    for c in range(_CHUNKS):
        if c >= 1 and c - 1 + _SLOTS < _CHUNKS:
            out_copy(c - 1).wait()
            in_copy(c - 1 + _SLOTS).start()
        in_copy(c).wait()
        out_copy(c).start()
    for c in range(max(_CHUNKS - _SLOTS, 0), _CHUNKS):
        out_copy(c).wait()


def kernel(prototypes):
    return pl.pallas_call(
        _copy_body,
        out_shape=jax.ShapeDtypeStruct((_ROWS, _COLS), prototypes.dtype),
        in_specs=[pl.BlockSpec(memory_space=pltpu.MemorySpace.HBM)],
        out_specs=pl.BlockSpec(memory_space=pltpu.MemorySpace.HBM),
        scratch_shapes=[
            pltpu.VMEM((_SLOTS, _CHUNK_ROWS, _COLS), jnp.float32),
            pltpu.SemaphoreType.DMA((_SLOTS,)),
            pltpu.SemaphoreType.DMA((_SLOTS,)),
        ],
    )(prototypes)
